# Initial kernel scaffold; baseline (speedup 1.0000x reference)
import jax
import jax.numpy as jnp
from jax.experimental import pallas as pl


def kernel(query, support, table, proj1_w, proj1_b, proj2_w, proj2_b, ln_a, ln_b, w_ih, w_hh, b_ih, b_hh):
    raise NotImplementedError("write your pallas kernel here")


# speedup vs baseline: 1.5801x; 1.5801x over previous
import functools

import jax
import jax.numpy as jnp
from jax import lax
from jax.experimental import pallas as pl
from jax.experimental.pallas import tpu as pltpu
from jax.experimental.pallas import tpu_sc as plsc

EMBED = 128
DM = 256
DI = 512
HID = 512
G4 = 4 * HID
STEPS = 4
B = 4096
FEW = 5
SUP_PAD = 8

NW = 32
CHUNK = 40
CHUNKS_PER_W = 8
B_PER_W = CHUNK * CHUNKS_PER_W
N_IDX = NW * B_PER_W


def _sc_gather_body(table_hbm, idx_hbm, out_hbm, idx_v, rows_v, sem):
    wid = lax.axis_index("s") * 2 + lax.axis_index("c")
    idx_base = pl.multiple_of(wid * CHUNKS_PER_W, 8)
    out_base = pl.multiple_of(wid * B_PER_W, 8)
    pltpu.sync_copy(idx_hbm.at[pl.ds(idx_base, CHUNKS_PER_W)], idx_v)
    for j in range(CHUNKS_PER_W):
        pltpu.async_copy(table_hbm.at[idx_v.at[j]],
                         rows_v.at[pl.ds(j * CHUNK, CHUNK)], sem).wait()
    pltpu.sync_copy(rows_v, out_hbm.at[pl.ds(out_base, B_PER_W)])


def _sc_gather(table, idx2d):
    mesh = plsc.VectorSubcoreMesh(core_axis_name="c", subcore_axis_name="s")
    return pl.kernel(
        _sc_gather_body,
        mesh=mesh,
        out_type=jax.ShapeDtypeStruct((N_IDX, EMBED), jnp.float32),
        scratch_types=[
            pltpu.VMEM((CHUNKS_PER_W, CHUNK), jnp.int32),
            pltpu.VMEM((B_PER_W, EMBED), jnp.float32),
            pltpu.SemaphoreType.DMA,
        ],
    )(table, idx2d)


BT = 256


def _dotT(a, b):
    return lax.dot_general(a, b, (((1,), (1,)), ((), ())),
                           preferred_element_type=jnp.float32)


def _tc_body(q_ref, s_ref, p1w_ref, p1b_ref, p2w_ref, p2b_ref, lna_ref,
             lnb_ref, wih_ref, whh1_ref, whh2_ref, bsum_ref, out_ref):
    s = s_ref[...]
    h1 = jnp.maximum(_dotT(s, p1w_ref[...]) + p1b_ref[...], 0.0)
    z = _dotT(h1, p2w_ref[...]) + p2b_ref[...] + s
    mu = jnp.mean(z, axis=-1, keepdims=True)
    var = jnp.sum((z - mu) ** 2, axis=-1, keepdims=True) / (DM - 1)
    sg = (z - mu) / (jnp.sqrt(var) + 1e-6) * lna_ref[...] + lnb_ref[...]
    row_ids = lax.broadcasted_iota(jnp.int32, (SUP_PAD, DM), 0)
    sg = jnp.where(row_ids < FEW, sg, 0.0)

    m = _dotT(sg, whh2_ref[...])

    q = q_ref[...]
    gq = _dotT(q, wih_ref[...]) + bsum_ref[...]

    col_ids = lax.broadcasted_iota(jnp.int32, (BT, SUP_PAD), 1)
    logit_mask = jnp.where(col_ids < FEW, 0.0, -1e30)

    c = jnp.zeros((BT, HID), jnp.float32)
    h = q
    gates = gq
    for step in range(STEPS):
        if step > 0:
            att = jax.nn.softmax(_dotT(h, sg) + logit_mask, axis=-1)
            gates = (gq + _dotT(h, whh1_ref[...])
                     + lax.dot_general(att, m, (((1,), (0,)), ((), ())),
                                       preferred_element_type=jnp.float32))
        i = jax.nn.sigmoid(gates[:, :HID])
        f = jax.nn.sigmoid(gates[:, HID:2 * HID])
        g = jnp.tanh(gates[:, 2 * HID:3 * HID])
        o = jax.nn.sigmoid(gates[:, 3 * HID:])
        c = f * c + i * g
        h = q + (o * jnp.tanh(c))[:, :DM]
    out_ref[...] = _dotT(h, sg)


def _tc_call(q, s_pad, p1w, p1b, p2w, p2b, lna, lnb, wih, whh1, whh2, bsum):
    full = lambda shape: pl.BlockSpec(shape, lambda i: (0, 0))
    return pl.pallas_call(
        _tc_body,
        grid=(B // BT,),
        in_specs=[
            pl.BlockSpec((BT, DM), lambda i: (i, 0)),
            full((SUP_PAD, DM)),
            full((DI, DM)),
            full((1, DI)),
            full((DM, DI)),
            full((1, DM)),
            full((1, DM)),
            full((1, DM)),
            full((G4, DM)),
            full((G4, DM)),
            full((G4, DM)),
            full((1, G4)),
        ],
        out_specs=pl.BlockSpec((BT, SUP_PAD), lambda i: (i, 0)),
        out_shape=jax.ShapeDtypeStruct((B, SUP_PAD), jnp.float32),
    )(q, s_pad, p1w, p1b, p2w, p2b, lna, lnb, wih, whh1, whh2, bsum)


def kernel(query, support, table, proj1_w, proj1_b, proj2_w, proj2_b,
           ln_a, ln_b, w_ih, w_hh, b_ih, b_hh):
    qi = query.reshape(-1).astype(jnp.int32)
    si = support.reshape(-1).astype(jnp.int32)
    pad = jnp.zeros((N_IDX - qi.shape[0] - si.shape[0],), jnp.int32)
    idx2d = jnp.concatenate([qi, si, pad]).reshape(NW * CHUNKS_PER_W, CHUNK)

    rows = _sc_gather(table, idx2d)
    q = rows[:2 * B].reshape(B, DM)
    s_pad = jnp.concatenate(
        [rows[2 * B:2 * B + 2 * FEW].reshape(FEW, DM),
         jnp.zeros((SUP_PAD - FEW, DM), jnp.float32)], axis=0)

    scores = _tc_call(
        q, s_pad, proj1_w, proj1_b.reshape(1, DI), proj2_w,
        proj2_b.reshape(1, DM), ln_a.reshape(1, DM), ln_b.reshape(1, DM),
        w_ih, w_hh[:, :DM], w_hh[:, DM:], (b_ih + b_hh).reshape(1, G4))
    return scores[:, :FEW]


# speedup vs baseline: 3.0418x; 1.9251x over previous
import functools

import jax
import jax.numpy as jnp
from jax import lax
from jax.experimental import pallas as pl
from jax.experimental.pallas import tpu as pltpu
from jax.experimental.pallas import tpu_sc as plsc

EMBED = 128
DM = 256
DI = 512
HID = 512
G4 = 4 * HID
STEPS = 4
B = 4096
FEW = 5
SUP_PAD = 8

NW = 32
CHUNK = 128
CHUNKS_PER_W = 2
B_PER_W = CHUNK * CHUNKS_PER_W
SUP_IDX = 16
N_IDX = NW * B_PER_W + SUP_IDX


def _sc_gather_body(table_hbm, idxq_hbm, idxs_hbm, out_hbm, idx_v, idxs_v,
                    rows_v, sem, sem_s):
    wid = lax.axis_index("s") * 2 + lax.axis_index("c")
    out_base = pl.multiple_of(wid * B_PER_W, 8)
    pltpu.sync_copy(idxq_hbm.at[wid], idx_v)
    gathers = [
        pltpu.async_copy(table_hbm.at[idx_v.at[j]],
                         rows_v.at[pl.ds(j * CHUNK, CHUNK)], sem)
        for j in range(CHUNKS_PER_W)
    ]

    @pl.when(wid == 0)
    def _():
        pltpu.sync_copy(idxs_hbm, idxs_v)
        pltpu.async_copy(table_hbm.at[idxs_v],
                         rows_v.at[pl.ds(B_PER_W, SUP_IDX)], sem_s).wait()
        pltpu.async_copy(rows_v.at[pl.ds(B_PER_W, SUP_IDX)],
                         out_hbm.at[pl.ds(NW * B_PER_W, SUP_IDX)],
                         sem_s).wait()

    for g in gathers:
        g.wait()
    pltpu.sync_copy(rows_v.at[pl.ds(0, B_PER_W)],
                    out_hbm.at[pl.ds(out_base, B_PER_W)])


def _sc_gather(table, idxq, idxs):
    mesh = plsc.VectorSubcoreMesh(core_axis_name="c", subcore_axis_name="s")
    return pl.kernel(
        _sc_gather_body,
        mesh=mesh,
        out_type=jax.ShapeDtypeStruct((N_IDX, EMBED), jnp.float32),
        scratch_types=[
            pltpu.VMEM((CHUNKS_PER_W, CHUNK), jnp.int32),
            pltpu.VMEM((SUP_IDX,), jnp.int32),
            pltpu.VMEM((B_PER_W + SUP_IDX, EMBED), jnp.float32),
            pltpu.SemaphoreType.DMA,
            pltpu.SemaphoreType.DMA,
        ],
    )(table, idxq, idxs)


BT = 256


def _dotT(a, b):
    return lax.dot_general(a, b, (((1,), (1,)), ((), ())),
                           preferred_element_type=jnp.float32)


def _tc_body(q_ref, s_ref, p1w_ref, p1b_ref, p2w_ref, p2b_ref, lna_ref,
             lnb_ref, wih_ref, whh1_ref, whh2_ref, bsum_ref, out_ref):
    s = s_ref[...]
    h1 = jnp.maximum(_dotT(s, p1w_ref[...]) + p1b_ref[...], 0.0)
    z = _dotT(h1, p2w_ref[...]) + p2b_ref[...] + s
    mu = jnp.mean(z, axis=-1, keepdims=True)
    var = jnp.sum((z - mu) ** 2, axis=-1, keepdims=True) / (DM - 1)
    sg = (z - mu) / (jnp.sqrt(var) + 1e-6) * lna_ref[...] + lnb_ref[...]
    row_ids = lax.broadcasted_iota(jnp.int32, (SUP_PAD, DM), 0)
    sg = jnp.where(row_ids < FEW, sg, 0.0)

    m = _dotT(sg, whh2_ref[...])

    q = q_ref[...]
    gq = _dotT(q, wih_ref[...]) + bsum_ref[...]

    col_ids = lax.broadcasted_iota(jnp.int32, (BT, SUP_PAD), 1)
    logit_mask = jnp.where(col_ids < FEW, 0.0, -1e30)

    c = jnp.zeros((BT, HID), jnp.float32)
    h = q
    gates = gq
    for step in range(STEPS):
        if step > 0:
            att = jax.nn.softmax(_dotT(h, sg) + logit_mask, axis=-1)
            gates = (gq + _dotT(h, whh1_ref[...])
                     + lax.dot_general(att, m, (((1,), (0,)), ((), ())),
                                       preferred_element_type=jnp.float32))
        i = jax.nn.sigmoid(gates[:, :HID])
        f = jax.nn.sigmoid(gates[:, HID:2 * HID])
        g = jnp.tanh(gates[:, 2 * HID:3 * HID])
        o = jax.nn.sigmoid(gates[:, 3 * HID:])
        c = f * c + i * g
        h = q + (o * jnp.tanh(c))[:, :DM]
    out_ref[...] = _dotT(h, sg)


def _tc_call(q, s_pad, p1w, p1b, p2w, p2b, lna, lnb, wih, whh1, whh2, bsum):
    full = lambda shape: pl.BlockSpec(shape, lambda i: (0, 0))
    return pl.pallas_call(
        _tc_body,
        grid=(B // BT,),
        in_specs=[
            pl.BlockSpec((BT, DM), lambda i: (i, 0)),
            full((SUP_PAD, DM)),
            full((DI, DM)),
            full((1, DI)),
            full((DM, DI)),
            full((1, DM)),
            full((1, DM)),
            full((1, DM)),
            full((G4, DM)),
            full((G4, DM)),
            full((G4, DM)),
            full((1, G4)),
        ],
        out_specs=pl.BlockSpec((BT, SUP_PAD), lambda i: (i, 0)),
        out_shape=jax.ShapeDtypeStruct((B, SUP_PAD), jnp.float32),
    )(q, s_pad, p1w, p1b, p2w, p2b, lna, lnb, wih, whh1, whh2, bsum)


def kernel(query, support, table, proj1_w, proj1_b, proj2_w, proj2_b,
           ln_a, ln_b, w_ih, w_hh, b_ih, b_hh):
    qi = query.reshape(-1).astype(jnp.int32)
    si = support.reshape(-1).astype(jnp.int32)
    idxq = qi.reshape(NW, CHUNKS_PER_W, CHUNK)
    idxs = jnp.concatenate(
        [si, jnp.zeros((SUP_IDX - si.shape[0],), jnp.int32)])

    rows = _sc_gather(table, idxq, idxs)
    q = rows[:2 * B].reshape(B, DM)
    s_pad = jnp.concatenate(
        [rows[2 * B:2 * B + 2 * FEW].reshape(FEW, DM),
         jnp.zeros((SUP_PAD - FEW, DM), jnp.float32)], axis=0)

    scores = _tc_call(
        q, s_pad, proj1_w, proj1_b.reshape(1, DI), proj2_w,
        proj2_b.reshape(1, DM), ln_a.reshape(1, DM), ln_b.reshape(1, DM),
        w_ih, w_hh[:, :DM], w_hh[:, DM:], (b_ih + b_hh).reshape(1, G4))
    return scores[:, :FEW]


# speedup vs baseline: 3.2323x; 1.0627x over previous
import functools

import jax
import jax.numpy as jnp
from jax import lax
from jax.experimental import pallas as pl
from jax.experimental.pallas import tpu as pltpu
from jax.experimental.pallas import tpu_sc as plsc

EMBED = 128
DM = 256
DI = 512
HID = 512
G4 = 4 * HID
STEPS = 4
B = 4096
FEW = 5
SUP_PAD = 8

NW = 32
CHUNK = 128
CHUNKS_PER_W = 2
B_PER_W = CHUNK * CHUNKS_PER_W
SUP_IDX = 16
N_IDX = NW * B_PER_W + SUP_IDX


def _sc_gather_body(table_hbm, idxq_hbm, idxs_hbm, out_hbm, idx_v, idxs_v,
                    rows_v, sem, sem_s):
    wid = lax.axis_index("s") * 2 + lax.axis_index("c")
    out_base = pl.multiple_of(wid * B_PER_W, 8)
    pltpu.sync_copy(idxq_hbm.at[wid], idx_v)
    gathers = [
        pltpu.async_copy(table_hbm.at[idx_v.at[j]],
                         rows_v.at[pl.ds(j * CHUNK, CHUNK)], sem)
        for j in range(CHUNKS_PER_W)
    ]

    @pl.when(wid == 0)
    def _():
        pltpu.sync_copy(idxs_hbm, idxs_v)
        pltpu.async_copy(table_hbm.at[idxs_v],
                         rows_v.at[pl.ds(B_PER_W, SUP_IDX)], sem_s).wait()
        pltpu.async_copy(rows_v.at[pl.ds(B_PER_W, SUP_IDX)],
                         out_hbm.at[pl.ds(NW * B_PER_W, SUP_IDX)],
                         sem_s).wait()

    for g in gathers:
        g.wait()
    pltpu.sync_copy(rows_v.at[pl.ds(0, B_PER_W)],
                    out_hbm.at[pl.ds(out_base, B_PER_W)])


def _sc_gather(table, idxq, idxs):
    mesh = plsc.VectorSubcoreMesh(core_axis_name="c", subcore_axis_name="s")
    return pl.kernel(
        _sc_gather_body,
        mesh=mesh,
        out_type=jax.ShapeDtypeStruct((N_IDX, EMBED), jnp.float32),
        scratch_types=[
            pltpu.VMEM((CHUNKS_PER_W, CHUNK), jnp.int32),
            pltpu.VMEM((SUP_IDX,), jnp.int32),
            pltpu.VMEM((B_PER_W + SUP_IDX, EMBED), jnp.float32),
            pltpu.SemaphoreType.DMA,
            pltpu.SemaphoreType.DMA,
        ],
    )(table, idxq, idxs)


BT = 256
GW = 4 * DM


def _dotT(a, b):
    return lax.dot_general(a, b, (((1,), (1,)), ((), ())),
                           preferred_element_type=jnp.float32)


def _tc_body(q_ref, s_ref, p1w_ref, p1b_ref, p2w_ref, p2b_ref, lna_ref,
             lnb_ref, wih_ref, whh1_ref, whh2_ref, bsum_ref, out_ref):
    s = jnp.reshape(s_ref[...], (SUP_PAD, DM))
    h1 = jnp.maximum(_dotT(s, p1w_ref[...]) + p1b_ref[...], 0.0)
    z = _dotT(h1, p2w_ref[...]) + p2b_ref[...] + s
    mu = jnp.mean(z, axis=-1, keepdims=True)
    var = jnp.sum((z - mu) ** 2, axis=-1, keepdims=True) / (DM - 1)
    sg = (z - mu) / (jnp.sqrt(var) + 1e-6) * lna_ref[...] + lnb_ref[...]
    row_ids = lax.broadcasted_iota(jnp.int32, (SUP_PAD, DM), 0)
    sg = jnp.where(row_ids < FEW, sg, 0.0)

    m = _dotT(sg, whh2_ref[...])

    q = jnp.reshape(q_ref[...], (BT, DM))
    gq = _dotT(q, wih_ref[...]) + bsum_ref[...]

    col_ids = lax.broadcasted_iota(jnp.int32, (BT, SUP_PAD), 1)
    logit_mask = jnp.where(col_ids < FEW, 0.0, -1e30)

    c = jnp.zeros((BT, DM), jnp.float32)
    h = q
    gates = gq
    for step in range(STEPS):
        if step > 0:
            att = jax.nn.softmax(_dotT(h, sg) + logit_mask, axis=-1)
            gates = (gq + _dotT(h, whh1_ref[...])
                     + lax.dot_general(att, m, (((1,), (0,)), ((), ())),
                                       preferred_element_type=jnp.float32))
        i = jax.nn.sigmoid(gates[:, :DM])
        f = jax.nn.sigmoid(gates[:, DM:2 * DM])
        g = jnp.tanh(gates[:, 2 * DM:3 * DM])
        o = jax.nn.sigmoid(gates[:, 3 * DM:])
        c = f * c + i * g
        h = q + o * jnp.tanh(c)
    out_ref[...] = _dotT(h, sg)[:, :FEW]


def _tc_call(rows, p1w, p1b, p2w, p2b, lna, lnb, wihx, whh1x, whh2x, bsumx):
    full = lambda shape: pl.BlockSpec(shape, lambda i: (0, 0))
    return pl.pallas_call(
        _tc_body,
        grid=(B // BT,),
        in_specs=[
            pl.BlockSpec((2 * BT, EMBED), lambda i: (i, 0)),
            pl.BlockSpec((2 * SUP_PAD, EMBED), lambda i: (2 * B // (2 * SUP_PAD), 0)),
            full((DI, DM)),
            full((1, DI)),
            full((DM, DI)),
            full((1, DM)),
            full((1, DM)),
            full((1, DM)),
            full((GW, DM)),
            full((GW, DM)),
            full((GW, DM)),
            full((1, GW)),
        ],
        out_specs=pl.BlockSpec((BT, FEW), lambda i: (i, 0)),
        out_shape=jax.ShapeDtypeStruct((B, FEW), jnp.float32),
    )(rows, rows, p1w, p1b, p2w, p2b, lna, lnb, wihx, whh1x, whh2x, bsumx)


def kernel(query, support, table, proj1_w, proj1_b, proj2_w, proj2_b,
           ln_a, ln_b, w_ih, w_hh, b_ih, b_hh):
    qi = query.reshape(-1).astype(jnp.int32)
    si = support.reshape(-1).astype(jnp.int32)
    idxq = qi.reshape(NW, CHUNKS_PER_W, CHUNK)
    idxs = jnp.concatenate(
        [si, jnp.zeros((SUP_IDX - si.shape[0],), jnp.int32)])

    rows = _sc_gather(table, idxq, idxs)

    wihx = w_ih.reshape(4, HID, DM)[:, :DM, :].reshape(GW, DM)
    whhx = w_hh.reshape(4, HID, 2 * DM)[:, :DM, :].reshape(GW, 2 * DM)
    bsumx = (b_ih + b_hh).reshape(4, HID)[:, :DM].reshape(1, GW)

    return _tc_call(
        rows, proj1_w, proj1_b.reshape(1, DI), proj2_w,
        proj2_b.reshape(1, DM), ln_a.reshape(1, DM), ln_b.reshape(1, DM),
        wihx, whhx[:, :DM], whhx[:, DM:], bsumx)


# speedup vs baseline: 4.4237x; 1.3686x over previous
import functools

import jax
import jax.numpy as jnp
from jax import lax
from jax.experimental import pallas as pl
from jax.experimental.pallas import tpu as pltpu
from jax.experimental.pallas import tpu_sc as plsc

EMBED = 128
DM = 256
DI = 512
HID = 512
G4 = 4 * HID
STEPS = 4
B = 4096
FEW = 5
SUP_PAD = 8

NW = 32
CHUNK = 128
CHUNKS_PER_W = 2
B_PER_W = CHUNK * CHUNKS_PER_W
SUP_IDX = 16
N_IDX = NW * B_PER_W + SUP_IDX


def _sc_gather_body(table_hbm, idxq_hbm, idxs_hbm, out_hbm, idx_v, idxs_v,
                    rows_v, sem, sem_s):
    wid = lax.axis_index("s") * 2 + lax.axis_index("c")
    out_base = pl.multiple_of(wid * B_PER_W, 8)
    pltpu.sync_copy(idxq_hbm.at[wid], idx_v)
    gathers = [
        pltpu.async_copy(table_hbm.at[idx_v.at[j]],
                         rows_v.at[pl.ds(j * CHUNK, CHUNK)], sem)
        for j in range(CHUNKS_PER_W)
    ]

    @pl.when(wid == 0)
    def _():
        pltpu.sync_copy(idxs_hbm, idxs_v)
        pltpu.async_copy(table_hbm.at[idxs_v],
                         rows_v.at[pl.ds(B_PER_W, SUP_IDX)], sem_s).wait()
        pltpu.async_copy(rows_v.at[pl.ds(B_PER_W, SUP_IDX)],
                         out_hbm.at[pl.ds(NW * B_PER_W, SUP_IDX)],
                         sem_s).wait()

    for g in gathers:
        g.wait()
    pltpu.sync_copy(rows_v.at[pl.ds(0, B_PER_W)],
                    out_hbm.at[pl.ds(out_base, B_PER_W)])


def _sc_gather(table, idxq, idxs):
    mesh = plsc.VectorSubcoreMesh(core_axis_name="c", subcore_axis_name="s")
    return pl.kernel(
        _sc_gather_body,
        mesh=mesh,
        out_type=jax.ShapeDtypeStruct((N_IDX, EMBED), jnp.float32),
        scratch_types=[
            pltpu.VMEM((CHUNKS_PER_W, CHUNK), jnp.int32),
            pltpu.VMEM((SUP_IDX,), jnp.int32),
            pltpu.VMEM((B_PER_W + SUP_IDX, EMBED), jnp.float32),
            pltpu.SemaphoreType.DMA,
            pltpu.SemaphoreType.DMA,
        ],
    )(table, idxq, idxs)


BT = 1024
GW = 4 * DM


def _dotT(a, b):
    return lax.dot_general(a, b, (((1,), (1,)), ((), ())),
                           preferred_element_type=jnp.float32)


def _tc_body(q_ref, s_ref, p1w_ref, p1b_ref, p2w_ref, p2b_ref, lna_ref,
             lnb_ref, wih_ref, whh1_ref, whh2_ref, bsum_ref, out_ref):
    s = jnp.reshape(s_ref[...], (SUP_PAD, DM))
    h1 = jnp.maximum(_dotT(s, p1w_ref[...]) + p1b_ref[...], 0.0)
    z = _dotT(h1, p2w_ref[...]) + p2b_ref[...] + s
    mu = jnp.mean(z, axis=-1, keepdims=True)
    var = jnp.sum((z - mu) ** 2, axis=-1, keepdims=True) / (DM - 1)
    sg = (z - mu) / (jnp.sqrt(var) + 1e-6) * lna_ref[...] + lnb_ref[...]
    row_ids = lax.broadcasted_iota(jnp.int32, (SUP_PAD, DM), 0)
    sg = jnp.where(row_ids < FEW, sg, 0.0)

    m = _dotT(sg, whh2_ref[...])

    q = jnp.reshape(q_ref[...], (BT, DM))
    gq = _dotT(q, wih_ref[...]) + bsum_ref[...]

    col_ids = lax.broadcasted_iota(jnp.int32, (BT, SUP_PAD), 1)
    logit_mask = jnp.where(col_ids < FEW, 0.0, -1e30)

    c = jnp.zeros((BT, DM), jnp.float32)
    h = q
    gates = gq
    for step in range(STEPS):
        if step > 0:
            att = jax.nn.softmax(_dotT(h, sg) + logit_mask, axis=-1)
            gates = (gq + _dotT(h, whh1_ref[...])
                     + lax.dot_general(att, m, (((1,), (0,)), ((), ())),
                                       preferred_element_type=jnp.float32))
        i = jax.nn.sigmoid(gates[:, :DM])
        f = jax.nn.sigmoid(gates[:, DM:2 * DM])
        g = jnp.tanh(gates[:, 2 * DM:3 * DM])
        o = jax.nn.sigmoid(gates[:, 3 * DM:])
        c = f * c + i * g
        h = q + o * jnp.tanh(c)
    out_ref[...] = _dotT(h, sg)[:, :FEW]


def _tc_call(rows, p1w, p1b, p2w, p2b, lna, lnb, wihx, whh1x, whh2x, bsumx):
    full = lambda shape: pl.BlockSpec(shape, lambda i: (0, 0))
    return pl.pallas_call(
        _tc_body,
        grid=(B // BT,),
        in_specs=[
            pl.BlockSpec((2 * BT, EMBED), lambda i: (i, 0)),
            pl.BlockSpec((2 * SUP_PAD, EMBED), lambda i: (2 * B // (2 * SUP_PAD), 0)),
            full((DI, DM)),
            full((1, DI)),
            full((DM, DI)),
            full((1, DM)),
            full((1, DM)),
            full((1, DM)),
            full((GW, DM)),
            full((GW, DM)),
            full((GW, DM)),
            full((1, GW)),
        ],
        out_specs=pl.BlockSpec((BT, FEW), lambda i: (i, 0)),
        out_shape=jax.ShapeDtypeStruct((B, FEW), jnp.float32),
    )(rows, rows, p1w, p1b, p2w, p2b, lna, lnb, wihx, whh1x, whh2x, bsumx)


def kernel(query, support, table, proj1_w, proj1_b, proj2_w, proj2_b,
           ln_a, ln_b, w_ih, w_hh, b_ih, b_hh):
    qi = query.reshape(-1).astype(jnp.int32)
    si = support.reshape(-1).astype(jnp.int32)
    idxq = qi.reshape(NW, CHUNKS_PER_W, CHUNK)
    idxs = jnp.concatenate(
        [si, jnp.zeros((SUP_IDX - si.shape[0],), jnp.int32)])

    rows = _sc_gather(table, idxq, idxs)

    wihx = w_ih.reshape(4, HID, DM)[:, :DM, :].reshape(GW, DM)
    whhx = w_hh.reshape(4, HID, 2 * DM)[:, :DM, :].reshape(GW, 2 * DM)
    bsumx = (b_ih + b_hh).reshape(4, HID)[:, :DM].reshape(1, GW)

    return _tc_call(
        rows, proj1_w, proj1_b.reshape(1, DI), proj2_w,
        proj2_b.reshape(1, DM), ln_a.reshape(1, DM), ln_b.reshape(1, DM),
        wihx, whhx[:, :DM], whhx[:, DM:], bsumx)


# speedup vs baseline: 4.5201x; 1.0218x over previous
import functools

import jax
import jax.numpy as jnp
from jax import lax
from jax.experimental import pallas as pl
from jax.experimental.pallas import tpu as pltpu
from jax.experimental.pallas import tpu_sc as plsc

EMBED = 128
DM = 256
DI = 512
HID = 512
G4 = 4 * HID
STEPS = 4
B = 4096
FEW = 5
SUP_PAD = 8

NW = 32
CHUNK = 128
CHUNKS_PER_W = 2
B_PER_W = CHUNK * CHUNKS_PER_W
SUP_IDX = 16
N_IDX = NW * B_PER_W + SUP_IDX


def _sc_gather_body(table_hbm, idxq_hbm, idxs_hbm, out_hbm, idx_v, idxs_v,
                    rows_v, sem, sem_s):
    wid = lax.axis_index("s") * 2 + lax.axis_index("c")
    out_base = pl.multiple_of(wid * B_PER_W, 8)
    pltpu.sync_copy(idxq_hbm.at[wid], idx_v)
    gathers = [
        pltpu.async_copy(table_hbm.at[idx_v.at[j]],
                         rows_v.at[pl.ds(j * CHUNK, CHUNK)], sem)
        for j in range(CHUNKS_PER_W)
    ]

    @pl.when(wid == 0)
    def _():
        pltpu.sync_copy(idxs_hbm, idxs_v)
        pltpu.async_copy(table_hbm.at[idxs_v],
                         rows_v.at[pl.ds(B_PER_W, SUP_IDX)], sem_s).wait()
        pltpu.async_copy(rows_v.at[pl.ds(B_PER_W, SUP_IDX)],
                         out_hbm.at[pl.ds(NW * B_PER_W, SUP_IDX)],
                         sem_s).wait()

    for g in gathers:
        g.wait()
    pltpu.sync_copy(rows_v.at[pl.ds(0, B_PER_W)],
                    out_hbm.at[pl.ds(out_base, B_PER_W)])


def _sc_gather(table, idxq, idxs):
    mesh = plsc.VectorSubcoreMesh(core_axis_name="c", subcore_axis_name="s")
    return pl.kernel(
        _sc_gather_body,
        mesh=mesh,
        out_type=jax.ShapeDtypeStruct((N_IDX, EMBED), jnp.float32),
        scratch_types=[
            pltpu.VMEM((CHUNKS_PER_W, CHUNK), jnp.int32),
            pltpu.VMEM((SUP_IDX,), jnp.int32),
            pltpu.VMEM((B_PER_W + SUP_IDX, EMBED), jnp.float32),
            pltpu.SemaphoreType.DMA,
            pltpu.SemaphoreType.DMA,
        ],
    )(table, idxq, idxs)


BT = 1024
GW = 4 * DM


def _dotT(a, b):
    return lax.dot_general(a, b, (((1,), (1,)), ((), ())),
                           preferred_element_type=jnp.float32)


def _tc_body(q_ref, s_ref, p1w_ref, p1b_ref, p2w_ref, p2b_ref, lna_ref,
             lnb_ref, wih_ref, whh_ref, bsum_ref, out_ref):
    wihx = jnp.reshape(wih_ref[...], (GW, DM))
    whhx = jnp.reshape(whh_ref[...], (GW, 2 * DM))
    whh1x = whhx[:, :DM]
    whh2x = whhx[:, DM:]
    bsumx = jnp.reshape(bsum_ref[...], (1, GW))
    s = jnp.reshape(s_ref[...], (SUP_PAD, DM))
    h1 = jnp.maximum(_dotT(s, p1w_ref[...]) + p1b_ref[...], 0.0)
    z = _dotT(h1, p2w_ref[...]) + p2b_ref[...] + s
    mu = jnp.mean(z, axis=-1, keepdims=True)
    var = jnp.sum((z - mu) ** 2, axis=-1, keepdims=True) / (DM - 1)
    sg = (z - mu) / (jnp.sqrt(var) + 1e-6) * lna_ref[...] + lnb_ref[...]
    row_ids = lax.broadcasted_iota(jnp.int32, (SUP_PAD, DM), 0)
    sg = jnp.where(row_ids < FEW, sg, 0.0)

    m = _dotT(sg, whh2x)

    q = jnp.reshape(q_ref[...], (BT, DM))
    gq = _dotT(q, wihx) + bsumx

    col_ids = lax.broadcasted_iota(jnp.int32, (BT, SUP_PAD), 1)
    logit_mask = jnp.where(col_ids < FEW, 0.0, -1e30)

    c = jnp.zeros((BT, DM), jnp.float32)
    h = q
    gates = gq
    for step in range(STEPS):
        if step > 0:
            att = jax.nn.softmax(_dotT(h, sg) + logit_mask, axis=-1)
            gates = (gq + _dotT(h, whh1x)
                     + lax.dot_general(att, m, (((1,), (0,)), ((), ())),
                                       preferred_element_type=jnp.float32))
        i = jax.nn.sigmoid(gates[:, :DM])
        f = jax.nn.sigmoid(gates[:, DM:2 * DM])
        g = jnp.tanh(gates[:, 2 * DM:3 * DM])
        o = jax.nn.sigmoid(gates[:, 3 * DM:])
        c = f * c + i * g
        h = q + o * jnp.tanh(c)
    out_ref[...] = _dotT(h, sg)[:, :FEW]


def _tc_call(rows, p1w, p1b, p2w, p2b, lna, lnb, wih3, whh3, bsum2):
    full = lambda shape: pl.BlockSpec(shape, lambda i: (0, 0))
    full3 = lambda shape: pl.BlockSpec(shape, lambda i: (0, 0, 0))
    return pl.pallas_call(
        _tc_body,
        grid=(B // BT,),
        in_specs=[
            pl.BlockSpec((2 * BT, EMBED), lambda i: (i, 0)),
            pl.BlockSpec((2 * SUP_PAD, EMBED), lambda i: (2 * B // (2 * SUP_PAD), 0)),
            full((DI, DM)),
            full((1, DI)),
            full((DM, DI)),
            full((1, DM)),
            full((1, DM)),
            full((1, DM)),
            full3((4, DM, DM)),
            full3((4, DM, 2 * DM)),
            full((4, DM)),
        ],
        out_specs=pl.BlockSpec((BT, FEW), lambda i: (i, 0)),
        out_shape=jax.ShapeDtypeStruct((B, FEW), jnp.float32),
    )(rows, rows, p1w, p1b, p2w, p2b, lna, lnb, wih3, whh3, bsum2)


def kernel(query, support, table, proj1_w, proj1_b, proj2_w, proj2_b,
           ln_a, ln_b, w_ih, w_hh, b_ih, b_hh):
    qi = query.reshape(-1).astype(jnp.int32)
    si = support.reshape(-1).astype(jnp.int32)
    idxq = qi.reshape(NW, CHUNKS_PER_W, CHUNK)
    idxs = jnp.concatenate(
        [si, jnp.zeros((SUP_IDX - si.shape[0],), jnp.int32)])

    rows = _sc_gather(table, idxq, idxs)

    return _tc_call(
        rows, proj1_w, proj1_b.reshape(1, DI), proj2_w,
        proj2_b.reshape(1, DM), ln_a.reshape(1, DM), ln_b.reshape(1, DM),
        w_ih.reshape(4, HID, DM), w_hh.reshape(4, HID, 2 * DM),
        (b_ih + b_hh).reshape(4, HID))
